# Initial kernel scaffold; baseline (speedup 1.0000x reference)
import jax
import jax.numpy as jnp
from jax.experimental import pallas as pl


def kernel(coords, xyz2, fmap1, fmap2, W1, b1, gamma, beta, prelu_a, W2, b2):
    raise NotImplementedError("write your pallas kernel here")


# speedup vs baseline: 15.7885x; 15.7885x over previous
import functools

import jax
import jax.numpy as jnp
from jax.experimental import pallas as pl
from jax.experimental.pallas import tpu as pltpu

N_P = 4096
NS = 8
M_BLK = 256
N_BLOCKS = N_P // M_BLK
BIG = float(N_P)


def _stage_a(coords_ref, xyz2_ref, fmap1_ref, fmap2_ref, w1_ref, b1_ref,
             feat_ref, part_ref):
    coords = coords_ref[0]
    xyz2 = xyz2_ref[0]
    f1 = fmap1_ref[0]
    f2 = fmap2_ref[0]

    sq_q = jnp.sum(coords * coords, axis=1, keepdims=True)
    sq_s = jnp.sum(xyz2 * xyz2, axis=1).reshape(1, N_P)
    cross = jax.lax.dot_general(coords, xyz2,
                                (((1,), (1,)), ((), ())))
    sqrdist = sq_q + sq_s - 2.0 * cross
    iota = jax.lax.broadcasted_iota(jnp.int32, (M_BLK, N_P), 1)
    vals = jnp.where(sqrdist <= 1.0, iota, N_P)

    corr = jax.lax.dot_general(f1, f2,
                               (((0,), (0,)), ((), ()))) * 0.125

    idxs = []
    for _ in range(NS):
        j = jnp.min(vals, axis=1, keepdims=True)
        idxs.append(j)
        vals = jnp.where(iota == j, N_P, vals)

    first = idxs[0]
    fixed = [jnp.minimum(jnp.where(j == N_P, first, j), N_P - 1)
             for j in idxs]

    corr_rows = []
    dxyz_rows = []
    for s in range(NS):
        onehot = (iota == fixed[s]).astype(jnp.float32)
        corr_rows.append(jnp.sum(onehot * corr, axis=1))
        gxyz = jnp.dot(onehot, xyz2)
        dxyz_rows.append(gxyz - coords)

    c0 = jnp.stack(corr_rows, axis=0)
    feat_ref[0, 0] = c0
    for c in range(3):
        feat_ref[0, 1 + c] = jnp.stack([d[:, c] for d in dxyz_rows], axis=0)

    feat_mat = jnp.concatenate(
        [c0.reshape(1, NS * M_BLK)] +
        [jnp.stack([d[:, c] for d in dxyz_rows], axis=0).reshape(1, NS * M_BLK)
         for c in range(3)], axis=0)
    x = jnp.dot(w1_ref[...], feat_mat) + b1_ref[...]
    sum_x = jnp.sum(x, axis=1)
    sum_x2 = jnp.sum(x * x, axis=1)
    part_ref[0, 0, 0] = sum_x
    part_ref[0, 0, 1] = sum_x2


def _stage_b(feat_ref, part_ref, w1_ref, b1_ref, gamma_ref, beta_ref,
             a_ref, w2_ref, b2_ref, out_ref):
    sum_x = jnp.sum(part_ref[0, :, 0, :], axis=0, keepdims=True)
    sum_x2 = jnp.sum(part_ref[0, :, 1, :], axis=0, keepdims=True)
    gi = jax.lax.broadcasted_iota(jnp.int32, (64, 64), 0) // 8
    gj = jax.lax.broadcasted_iota(jnp.int32, (64, 64), 1) // 8
    gmat = (gi == gj).astype(jnp.float32)
    n_tot = float(N_P * NS * 8)
    mean_c = jnp.dot(sum_x, gmat) / n_tot
    ex2_c = jnp.dot(sum_x2, gmat) / n_tot
    var_c = ex2_c - mean_c * mean_c
    inv_c = jax.lax.rsqrt(var_c + 1e-5)
    scale = gamma_ref[...] * inv_c
    shift = beta_ref[...] - mean_c * scale

    feat = feat_ref[0].reshape(4, NS * M_BLK)
    xt = jax.lax.dot_general(feat, w1_ref[...],
                             (((0,), (1,)), ((), ())))
    xt = xt + b1_ref[...]
    xt = xt * scale + shift
    a = a_ref[0, 0]
    xt = jnp.where(xt >= 0.0, xt, a * xt)
    mx = jnp.max(xt.reshape(NS, M_BLK, 64), axis=0)
    out = jax.lax.dot_general(w2_ref[...], mx,
                              (((1,), (1,)), ((), ())))
    out_ref[0] = out + b2_ref[...]


@jax.jit
def kernel(coords, xyz2, fmap1, fmap2, W1, b1, gamma, beta, prelu_a, W2, b2):
    b = coords.shape[0]
    b1c = b1.reshape(64, 1)
    b1r = b1.reshape(1, 64)
    gammar = gamma.reshape(1, 64)
    betar = beta.reshape(1, 64)
    b2c = b2.reshape(64, 1)
    ar = prelu_a.reshape(1, 1)

    grid = (b, N_BLOCKS)
    feat, part = pl.pallas_call(
        _stage_a,
        grid=grid,
        in_specs=[
            pl.BlockSpec((1, M_BLK, 3), lambda bi, mi: (bi, mi, 0)),
            pl.BlockSpec((1, N_P, 3), lambda bi, mi: (bi, 0, 0)),
            pl.BlockSpec((1, 64, M_BLK), lambda bi, mi: (bi, 0, mi)),
            pl.BlockSpec((1, 64, N_P), lambda bi, mi: (bi, 0, 0)),
            pl.BlockSpec((64, 4), lambda bi, mi: (0, 0)),
            pl.BlockSpec((64, 1), lambda bi, mi: (0, 0)),
        ],
        out_specs=[
            pl.BlockSpec((1, 4, NS, M_BLK), lambda bi, mi: (bi, 0, 0, mi)),
            pl.BlockSpec((1, 1, 2, 64), lambda bi, mi: (bi, mi, 0, 0)),
        ],
        out_shape=[
            jax.ShapeDtypeStruct((b, 4, NS, N_P), jnp.float32),
            jax.ShapeDtypeStruct((b, N_BLOCKS, 2, 64), jnp.float32),
        ],
        compiler_params=pltpu.CompilerParams(
            dimension_semantics=("parallel", "parallel")),
    )(coords, xyz2, fmap1, fmap2, W1, b1c)

    out = pl.pallas_call(
        _stage_b,
        grid=grid,
        in_specs=[
            pl.BlockSpec((1, 4, NS, M_BLK), lambda bi, mi: (bi, 0, 0, mi)),
            pl.BlockSpec((1, N_BLOCKS, 2, 64), lambda bi, mi: (bi, 0, 0, 0)),
            pl.BlockSpec((64, 4), lambda bi, mi: (0, 0)),
            pl.BlockSpec((1, 64), lambda bi, mi: (0, 0)),
            pl.BlockSpec((1, 64), lambda bi, mi: (0, 0)),
            pl.BlockSpec((1, 64), lambda bi, mi: (0, 0)),
            pl.BlockSpec((1, 1), lambda bi, mi: (0, 0)),
            pl.BlockSpec((64, 64), lambda bi, mi: (0, 0)),
            pl.BlockSpec((64, 1), lambda bi, mi: (0, 0)),
        ],
        out_specs=pl.BlockSpec((1, 64, M_BLK), lambda bi, mi: (bi, 0, mi)),
        out_shape=jax.ShapeDtypeStruct((b, 64, N_P), jnp.float32),
        compiler_params=pltpu.CompilerParams(
            dimension_semantics=("parallel", "parallel")),
    )(feat, part, W1, b1r, gammar, betar, ar, W2, b2c)
    return out


# speedup vs baseline: 46.3188x; 2.9337x over previous
import functools

import jax
import jax.numpy as jnp
from jax.experimental import pallas as pl
from jax.experimental.pallas import tpu as pltpu

N_P = 4096
NS = 8
M_BLK = 256
N_BLOCKS = N_P // M_BLK
BIG = float(N_P)


CHUNK = 256


def _select_extract(coords, xyz2, f1, w1, b1, sq_q, sqrdist, width, f2w,
                    need_fix, feat_ref, part_ref):
    iota = jax.lax.broadcasted_iota(jnp.int32, (M_BLK, width), 1)
    vals = jnp.where(sqrdist <= 1.0, iota, N_P)

    corr = jax.lax.dot_general(f1, f2w,
                               (((0,), (0,)), ((), ()))) * 0.125

    idxs = []
    for _ in range(NS):
        j = jnp.min(vals, axis=1, keepdims=True)
        idxs.append(j)
        vals = jnp.where(iota == j, N_P, vals)

    if need_fix:
        first = idxs[0]
        idxs = [jnp.minimum(jnp.where(j == N_P, first, j), N_P - 1)
                for j in idxs]

    xyz2w = xyz2[:width]
    corr_rows = []
    dxyz_rows = []
    for s in range(NS):
        onehot = (iota == idxs[s]).astype(jnp.float32)
        corr_rows.append(jnp.sum(onehot * corr, axis=1))
        gxyz = jnp.dot(onehot, xyz2w)
        dxyz_rows.append(gxyz - coords)

    c0 = jnp.stack(corr_rows, axis=0)
    feat_ref[0, 0] = c0
    dstk = [jnp.stack([d[:, c] for d in dxyz_rows], axis=0) for c in range(3)]
    for c in range(3):
        feat_ref[0, 1 + c] = dstk[c]

    feat_mat = jnp.concatenate(
        [c0.reshape(1, NS * M_BLK)] +
        [d.reshape(1, NS * M_BLK) for d in dstk], axis=0)
    x = jnp.dot(w1, feat_mat) + b1
    part_ref[0, 0, 0] = jnp.sum(x, axis=1)
    part_ref[0, 0, 1] = jnp.sum(x * x, axis=1)


def _stage_a(coords_ref, xyz2_ref, fmap1_ref, fmap2_ref, w1_ref, b1_ref,
             feat_ref, part_ref):
    coords = coords_ref[0]
    xyz2 = xyz2_ref[0]
    f1 = fmap1_ref[0]
    f2 = fmap2_ref[0]
    w1 = w1_ref[...]
    b1 = b1_ref[...]

    sq_q = jnp.sum(coords * coords, axis=1, keepdims=True)

    xyz2c = xyz2[:CHUNK]
    sq_sc = jnp.sum(xyz2c * xyz2c, axis=1).reshape(1, CHUNK)
    crossc = jax.lax.dot_general(coords, xyz2c,
                                 (((1,), (1,)), ((), ())))
    sqrdc = sq_q + sq_sc - 2.0 * crossc
    cnt = jnp.sum((sqrdc <= 1.0).astype(jnp.int32), axis=1)
    all_found = jnp.min(cnt) >= NS

    @pl.when(all_found)
    def _fast():
        _select_extract(coords, xyz2, f1, w1, b1, sq_q, sqrdc, CHUNK,
                        f2[:, :CHUNK], False, feat_ref, part_ref)

    @pl.when(jnp.logical_not(all_found))
    def _full():
        sq_s = jnp.sum(xyz2 * xyz2, axis=1).reshape(1, N_P)
        cross = jax.lax.dot_general(coords, xyz2,
                                    (((1,), (1,)), ((), ())))
        sqrdist = sq_q + sq_s - 2.0 * cross
        _select_extract(coords, xyz2, f1, w1, b1, sq_q, sqrdist, N_P,
                        f2, True, feat_ref, part_ref)


def _stage_b(feat_ref, part_ref, w1_ref, b1_ref, gamma_ref, beta_ref,
             a_ref, w2_ref, b2_ref, out_ref):
    sum_x = jnp.sum(part_ref[0, :, 0, :], axis=0, keepdims=True)
    sum_x2 = jnp.sum(part_ref[0, :, 1, :], axis=0, keepdims=True)
    gi = jax.lax.broadcasted_iota(jnp.int32, (64, 64), 0) // 8
    gj = jax.lax.broadcasted_iota(jnp.int32, (64, 64), 1) // 8
    gmat = (gi == gj).astype(jnp.float32)
    n_tot = float(N_P * NS * 8)
    mean_c = jnp.dot(sum_x, gmat) / n_tot
    ex2_c = jnp.dot(sum_x2, gmat) / n_tot
    var_c = ex2_c - mean_c * mean_c
    inv_c = jax.lax.rsqrt(var_c + 1e-5)
    scale = gamma_ref[...] * inv_c
    shift = beta_ref[...] - mean_c * scale

    feat = feat_ref[0].reshape(4, NS * M_BLK)
    xt = jax.lax.dot_general(feat, w1_ref[...],
                             (((0,), (1,)), ((), ())))
    xt = xt + b1_ref[...]
    xt = xt * scale + shift
    a = a_ref[0, 0]
    xt = jnp.where(xt >= 0.0, xt, a * xt)
    mx = jnp.max(xt.reshape(NS, M_BLK, 64), axis=0)
    out = jax.lax.dot_general(w2_ref[...], mx,
                              (((1,), (1,)), ((), ())))
    out_ref[0] = out + b2_ref[...]


@jax.jit
def kernel(coords, xyz2, fmap1, fmap2, W1, b1, gamma, beta, prelu_a, W2, b2):
    b = coords.shape[0]
    b1c = b1.reshape(64, 1)
    b1r = b1.reshape(1, 64)
    gammar = gamma.reshape(1, 64)
    betar = beta.reshape(1, 64)
    b2c = b2.reshape(64, 1)
    ar = prelu_a.reshape(1, 1)

    grid = (b, N_BLOCKS)
    feat, part = pl.pallas_call(
        _stage_a,
        grid=grid,
        in_specs=[
            pl.BlockSpec((1, M_BLK, 3), lambda bi, mi: (bi, mi, 0)),
            pl.BlockSpec((1, N_P, 3), lambda bi, mi: (bi, 0, 0)),
            pl.BlockSpec((1, 64, M_BLK), lambda bi, mi: (bi, 0, mi)),
            pl.BlockSpec((1, 64, N_P), lambda bi, mi: (bi, 0, 0)),
            pl.BlockSpec((64, 4), lambda bi, mi: (0, 0)),
            pl.BlockSpec((64, 1), lambda bi, mi: (0, 0)),
        ],
        out_specs=[
            pl.BlockSpec((1, 4, NS, M_BLK), lambda bi, mi: (bi, 0, 0, mi)),
            pl.BlockSpec((1, 1, 2, 64), lambda bi, mi: (bi, mi, 0, 0)),
        ],
        out_shape=[
            jax.ShapeDtypeStruct((b, 4, NS, N_P), jnp.float32),
            jax.ShapeDtypeStruct((b, N_BLOCKS, 2, 64), jnp.float32),
        ],
        compiler_params=pltpu.CompilerParams(
            dimension_semantics=("parallel", "parallel")),
    )(coords, xyz2, fmap1, fmap2, W1, b1c)

    out = pl.pallas_call(
        _stage_b,
        grid=grid,
        in_specs=[
            pl.BlockSpec((1, 4, NS, M_BLK), lambda bi, mi: (bi, 0, 0, mi)),
            pl.BlockSpec((1, N_BLOCKS, 2, 64), lambda bi, mi: (bi, 0, 0, 0)),
            pl.BlockSpec((64, 4), lambda bi, mi: (0, 0)),
            pl.BlockSpec((1, 64), lambda bi, mi: (0, 0)),
            pl.BlockSpec((1, 64), lambda bi, mi: (0, 0)),
            pl.BlockSpec((1, 64), lambda bi, mi: (0, 0)),
            pl.BlockSpec((1, 1), lambda bi, mi: (0, 0)),
            pl.BlockSpec((64, 64), lambda bi, mi: (0, 0)),
            pl.BlockSpec((64, 1), lambda bi, mi: (0, 0)),
        ],
        out_specs=pl.BlockSpec((1, 64, M_BLK), lambda bi, mi: (bi, 0, mi)),
        out_shape=jax.ShapeDtypeStruct((b, 64, N_P), jnp.float32),
        compiler_params=pltpu.CompilerParams(
            dimension_semantics=("parallel", "parallel")),
    )(feat, part, W1, b1r, gammar, betar, ar, W2, b2c)
    return out


# speedup vs baseline: 67.6123x; 1.4597x over previous
import jax
import jax.numpy as jnp
from jax.experimental import pallas as pl
from jax.experimental.pallas import tpu as pltpu

N_P = 4096
NS = 8
M_BLK = 256
N_BLOCKS = N_P // M_BLK
M2_BLK = 1024
N_BLOCKS2 = N_P // M2_BLK
CHUNK = 128


def _select_extract(coords_t, f1, w1, b1, sqrdist, width, table,
                    need_fix, feat_ref, part_ref):
    iota = jax.lax.broadcasted_iota(jnp.int32, (M_BLK, width), 1)
    vals = jnp.where(sqrdist <= 1.0, iota, N_P)

    idxs = []
    for _ in range(NS):
        j = jnp.min(vals, axis=1, keepdims=True)
        idxs.append(j)
        vals = jnp.where(iota == j, N_P, vals)

    if need_fix:
        first = idxs[0]
        idxs = [jnp.minimum(jnp.where(j == N_P, first, j), N_P - 1)
                for j in idxs]

    onehot_all = jnp.concatenate(
        [(iota == j).astype(jnp.float32) for j in idxs], axis=0)
    g_all = jax.lax.dot_general(table, onehot_all,
                                (((0,), (1,)), ((), ())))
    f1_rep = jnp.concatenate([f1] * NS, axis=1)
    corr_all = jnp.sum(f1_rep * g_all[:64], axis=0,
                       keepdims=True) * 0.125
    coords_rep = jnp.concatenate([coords_t] * NS, axis=1)
    dxyz_all = g_all[64:67] - coords_rep
    feat_all = jnp.concatenate([corr_all, dxyz_all], axis=0)

    for s in range(NS):
        feat_ref[0, :, s, :] = feat_all[:, s * M_BLK:(s + 1) * M_BLK]

    x = jnp.dot(w1, feat_all) + b1
    part_ref[0, 0, 0] = jnp.sum(x, axis=1)
    part_ref[0, 0, 1] = jnp.sum(x * x, axis=1)


def _stage_a(coords_ref, coords_t_ref, xyz2_ref, fmap1_ref, f2t_ref,
             w1_ref, b1_ref, feat_ref, part_ref):
    coords = coords_ref[0]
    coords_t = coords_t_ref[0]
    xyz2 = xyz2_ref[0]
    f1 = fmap1_ref[0]
    f2t = f2t_ref[0]
    w1 = w1_ref[...]
    b1 = b1_ref[...]

    sq_q = jnp.sum(coords * coords, axis=1, keepdims=True)

    xyz2c = xyz2[:CHUNK]
    sq_sc = jnp.sum(xyz2c * xyz2c, axis=1).reshape(1, CHUNK)
    crossc = jax.lax.dot_general(coords, xyz2c,
                                 (((1,), (1,)), ((), ())))
    sqrdc = sq_q + sq_sc - 2.0 * crossc
    cnt = jnp.sum((sqrdc <= 1.0).astype(jnp.int32), axis=1)
    all_found = jnp.min(cnt) >= NS

    @pl.when(all_found)
    def _fast():
        table = jnp.concatenate([f2t[:CHUNK], xyz2c], axis=1)
        _select_extract(coords_t, f1, w1, b1, sqrdc, CHUNK, table,
                        False, feat_ref, part_ref)

    @pl.when(jnp.logical_not(all_found))
    def _full():
        sq_s = jnp.sum(xyz2 * xyz2, axis=1).reshape(1, N_P)
        cross = jax.lax.dot_general(coords, xyz2,
                                    (((1,), (1,)), ((), ())))
        sqrdist = sq_q + sq_s - 2.0 * cross
        table = jnp.concatenate([f2t, xyz2], axis=1)
        _select_extract(coords_t, f1, w1, b1, sqrdist, N_P, table,
                        True, feat_ref, part_ref)


def _stage_b(feat_ref, part_ref, w1_ref, b1_ref, gamma_ref, beta_ref,
             a_ref, w2_ref, b2_ref, out_ref):
    sum_x = jnp.sum(part_ref[0, :, 0, :], axis=0, keepdims=True)
    sum_x2 = jnp.sum(part_ref[0, :, 1, :], axis=0, keepdims=True)
    gi = jax.lax.broadcasted_iota(jnp.int32, (64, 64), 0) // 8
    gj = jax.lax.broadcasted_iota(jnp.int32, (64, 64), 1) // 8
    gmat = (gi == gj).astype(jnp.float32)
    n_tot = float(N_P * NS * 8)
    mean_c = jnp.dot(sum_x, gmat) / n_tot
    ex2_c = jnp.dot(sum_x2, gmat) / n_tot
    var_c = ex2_c - mean_c * mean_c
    inv_c = jax.lax.rsqrt(var_c + 1e-5)
    scale = gamma_ref[...] * inv_c
    shift = beta_ref[...] - mean_c * scale

    feat = feat_ref[0].reshape(4, NS * M2_BLK)
    xt = jax.lax.dot_general(feat, w1_ref[...],
                             (((0,), (1,)), ((), ())))
    xt = xt + b1_ref[...]
    xt = xt * scale + shift
    a = a_ref[0, 0]
    xt = jnp.where(xt >= 0.0, xt, a * xt)
    mx = jnp.max(xt.reshape(NS, M2_BLK, 64), axis=0)
    out = jax.lax.dot_general(w2_ref[...], mx,
                              (((1,), (1,)), ((), ())))
    out_ref[0] = out + b2_ref[...]


@jax.jit
def kernel(coords, xyz2, fmap1, fmap2, W1, b1, gamma, beta, prelu_a, W2, b2):
    b = coords.shape[0]
    coords_t = jnp.transpose(coords, (0, 2, 1))
    f2t = jnp.transpose(fmap2, (0, 2, 1))
    b1c = b1.reshape(64, 1)
    b1r = b1.reshape(1, 64)
    gammar = gamma.reshape(1, 64)
    betar = beta.reshape(1, 64)
    b2c = b2.reshape(64, 1)
    ar = prelu_a.reshape(1, 1)

    feat, part = pl.pallas_call(
        _stage_a,
        grid=(b, N_BLOCKS),
        in_specs=[
            pl.BlockSpec((1, M_BLK, 3), lambda bi, mi: (bi, mi, 0)),
            pl.BlockSpec((1, 3, M_BLK), lambda bi, mi: (bi, 0, mi)),
            pl.BlockSpec((1, N_P, 3), lambda bi, mi: (bi, 0, 0)),
            pl.BlockSpec((1, 64, M_BLK), lambda bi, mi: (bi, 0, mi)),
            pl.BlockSpec((1, N_P, 64), lambda bi, mi: (bi, 0, 0)),
            pl.BlockSpec((64, 4), lambda bi, mi: (0, 0)),
            pl.BlockSpec((64, 1), lambda bi, mi: (0, 0)),
        ],
        out_specs=[
            pl.BlockSpec((1, 4, NS, M_BLK), lambda bi, mi: (bi, 0, 0, mi)),
            pl.BlockSpec((1, 1, 2, 64), lambda bi, mi: (bi, mi, 0, 0)),
        ],
        out_shape=[
            jax.ShapeDtypeStruct((b, 4, NS, N_P), jnp.float32),
            jax.ShapeDtypeStruct((b, N_BLOCKS, 2, 64), jnp.float32),
        ],
        compiler_params=pltpu.CompilerParams(
            dimension_semantics=("parallel", "parallel")),
    )(coords, coords_t, xyz2, fmap1, f2t, W1, b1c)

    out = pl.pallas_call(
        _stage_b,
        grid=(b, N_BLOCKS2),
        in_specs=[
            pl.BlockSpec((1, 4, NS, M2_BLK), lambda bi, mi: (bi, 0, 0, mi)),
            pl.BlockSpec((1, N_BLOCKS, 2, 64), lambda bi, mi: (bi, 0, 0, 0)),
            pl.BlockSpec((64, 4), lambda bi, mi: (0, 0)),
            pl.BlockSpec((1, 64), lambda bi, mi: (0, 0)),
            pl.BlockSpec((1, 64), lambda bi, mi: (0, 0)),
            pl.BlockSpec((1, 64), lambda bi, mi: (0, 0)),
            pl.BlockSpec((1, 1), lambda bi, mi: (0, 0)),
            pl.BlockSpec((64, 64), lambda bi, mi: (0, 0)),
            pl.BlockSpec((64, 1), lambda bi, mi: (0, 0)),
        ],
        out_specs=pl.BlockSpec((1, 64, M2_BLK), lambda bi, mi: (bi, 0, mi)),
        out_shape=jax.ShapeDtypeStruct((b, 64, N_P), jnp.float32),
        compiler_params=pltpu.CompilerParams(
            dimension_semantics=("parallel", "parallel")),
    )(feat, part, W1, b1r, gammar, betar, ar, W2, b2c)
    return out
